# baseline (speedup 1.0000x reference)
import functools

import jax
import jax.numpy as jnp
from jax import lax
from jax.experimental import pallas as pl
from jax.experimental.pallas import tpu as pltpu
from jax.experimental.pallas import tpu_sc as plsc

N = 10000
D = 256
E = 160000
KITER = 3
LAM1 = 3.0
GAMMA = 1.0 / (1.0 + 3.0)
BETA = 1.0 / (2.0 * GAMMA)

NSC = 16
HALF = 128
CH = 64
RT = 640
NP = NSC * RT
NDCH = RT // CH
DUMMY = N

E1 = E + N
E1C = -(-E1 // (NSC * CH))
E1T = E1C * CH
E1P = NSC * E1T
CH2 = 128
EC2 = -(-E // (NSC * CH2))
ET = EC2 * CH2
EP = NSC * ET
EC3 = ET // CH

_f32 = jnp.float32


def _rsqrt16(x):
    xi = lax.bitcast_convert_type(x, jnp.int32)
    yi = jnp.int32(0x5F3759DF) - (xi >> 1)
    y = lax.bitcast_convert_type(yi, _f32)
    for _ in range(4):
        y = y * (1.5 - 0.5 * x * y * y)
    return y


def _lanesum(v):
    lanes = lax.iota(jnp.int32, 16)
    for s in (8, 4, 2, 1):
        v = v + v.at[lanes ^ s].get(mode="promise_in_bounds")
    return v


def _mesh():
    return plsc.VectorSubcoreMesh(core_axis_name="c", subcore_axis_name="s")


def _k0_body(dstl, feat2, dinv_out, xs_out, accd, didx, fb, xb, db):
    c = lax.axis_index("c")
    t = lax.axis_index("s")
    r0 = t * RT
    def fill(i, _):
        for k in range(HALF // 16):
            sl = pl.ds(k * 16, 16)
            fb[i, sl] = jnp.zeros((16,), _f32)
            xb[i, sl] = jnp.ones((16,), _f32)
        return _
    lax.fori_loop(0, CH, fill, None)
    def zloop(j, _):
        pltpu.sync_copy(fb, accd.at[pl.ds(r0 + j * CH, CH)])
        return _
    lax.fori_loop(0, NDCH, zloop, None)
    plsc.subcore_barrier()
    def sloop(j, _):
        b = t * E1T + j * CH
        pltpu.sync_copy(dstl.at[pl.ds(b, CH)], didx)
        pltpu.sync_copy(xb, accd.at[didx], add=True)
        return _
    lax.fori_loop(0, E1C, sloop, None)
    plsc.subcore_barrier()
    def rloop(j, _):
        ridx = r0 + j * CH
        pltpu.sync_copy(accd.at[pl.ds(ridx, CH)], fb)
        def one(i, _2):
            d = jnp.maximum(fb[i, pl.ds(0, 16)], 1.0)
            db[i, :] = _rsqrt16(d)
            return _2
        lax.fori_loop(0, CH, one, None)
        pltpu.sync_copy(db, dinv_out.at[c].at[pl.ds(ridx, CH)])
        return _
    lax.fori_loop(0, NDCH, rloop, None)

    def xloop(j, _):
        ridx = r0 + j * CH
        pltpu.sync_copy(feat2.at[c].at[pl.ds(ridx, CH)], fb)
        pltpu.sync_copy(dinv_out.at[c].at[pl.ds(ridx, CH)], db)
        def one(i, _2):
            dv = db[i, :]
            for k in range(HALF // 16):
                sl = pl.ds(k * 16, 16)
                xb[i, sl] = dv * fb[i, sl]
            return _2
        lax.fori_loop(0, CH, one, None)
        pltpu.sync_copy(xb, xs_out.at[c].at[pl.ds(ridx, CH)])
        return _
    lax.fori_loop(0, NDCH, xloop, None)


@functools.lru_cache(maxsize=None)
def _k0():
    return pl.kernel(
        _k0_body,
        out_type=(
            jax.ShapeDtypeStruct((2, NP, 16), _f32),
            jax.ShapeDtypeStruct((2, NP, HALF), _f32),
        ),
        mesh=_mesh(),
        scratch_types=(
            pltpu.VMEM_SHARED((NP, HALF), _f32),
            pltpu.VMEM((CH,), jnp.int32),
            pltpu.VMEM((CH, HALF), _f32),
            pltpu.VMEM((CH, HALF), _f32),
            pltpu.VMEM((CH, 16), _f32),
        ),
    )


def _k1_body(xs, feat2, itz, dinv, zrows, srcl, dstl, y_out, xbs_out,
             acc, sidx, didx, ab, fb, ib, db, sem):
    c = lax.axis_index("c")
    t = lax.axis_index("s")
    r0 = t * RT
    pltpu.sync_copy(zrows.at[pl.ds(r0, RT)], acc.at[pl.ds(r0, RT)])
    plsc.subcore_barrier()

    def chunk(j, _):
        b = t * E1T + j * CH
        pltpu.sync_copy(srcl.at[pl.ds(b, CH)], sidx)
        pltpu.sync_copy(dstl.at[pl.ds(b, CH)], didx)
        pltpu.async_copy(xs.at[c].at[sidx], ab, sem).wait()
        pltpu.sync_copy(ab, acc.at[didx], add=True)
        return _
    lax.fori_loop(0, E1C, chunk, None)
    plsc.subcore_barrier()

    def dense(j, _):
        r = r0 + j * CH
        pltpu.sync_copy(acc.at[pl.ds(r, CH)], ab)
        pltpu.sync_copy(feat2.at[c].at[pl.ds(r, CH)], fb)
        pltpu.sync_copy(itz.at[c].at[pl.ds(r, CH)], ib)
        pltpu.sync_copy(dinv.at[pl.ds(r, CH)], db)
        @plsc.parallel_loop(0, CH, unroll=2)
        def one(i):
            dv = db[i, :]
            gdd = (GAMMA * dv) * dv
            for k in range(HALF // 16):
                sl = pl.ds(k * 16, 16)
                yv = GAMMA * fb[i, sl] + (1.0 - GAMMA) * (dv * ab[i, sl])
                fb[i, sl] = yv
                ab[i, sl] = dv * yv - gdd * ib[i, sl]
        pltpu.sync_copy(fb, y_out.at[c].at[pl.ds(r, CH)])
        pltpu.sync_copy(ab, xbs_out.at[c].at[pl.ds(r, CH)])
        return _
    lax.fori_loop(0, NDCH, dense, None)


@functools.lru_cache(maxsize=None)
def _k1():
    return pl.kernel(
        _k1_body,
        out_type=(
            jax.ShapeDtypeStruct((2, NP, HALF), _f32),
            jax.ShapeDtypeStruct((2, NP, HALF), _f32),
        ),
        mesh=_mesh(),
        scratch_types=(
            pltpu.VMEM_SHARED((NP, HALF), _f32),
            pltpu.VMEM((CH,), jnp.int32),
            pltpu.VMEM((CH,), jnp.int32),
            pltpu.VMEM((CH, HALF), _f32),
            pltpu.VMEM((CH, HALF), _f32),
            pltpu.VMEM((CH, HALF), _f32),
            pltpu.VMEM((CH, 16), _f32),
            pltpu.SemaphoreType.DMA,
        ),
    )


def _k2_body(xbs, z, rowp, colp, bm, zb_out, ps_out,
             ridx, cidx, ab, bb, zbuf, psb, bmb, sem, sem2):
    c = lax.axis_index("c")
    t = lax.axis_index("s")

    def chunk(j, _):
        b = t * ET + j * CH2
        pltpu.sync_copy(rowp.at[pl.ds(b, CH2)], ridx)
        pltpu.sync_copy(colp.at[pl.ds(b, CH2)], cidx)
        cpa = pltpu.async_copy(xbs.at[c].at[ridx], ab, sem)
        cpb = pltpu.async_copy(xbs.at[c].at[cidx], bb, sem2)
        pltpu.sync_copy(z.at[c].at[pl.ds(b, CH2)], zbuf)
        pltpu.sync_copy(bm.at[pl.ds(b, CH2)], bmb)
        cpa.wait()
        cpb.wait()
        @plsc.parallel_loop(0, CH2, unroll=2)
        def one(i):
            mv = bmb[i, :]
            accv = jnp.zeros((16,), _f32)
            for k in range(HALF // 16):
                sl = pl.ds(k * 16, 16)
                zv = zbuf[i, sl] + mv * (ab[i, sl] - bb[i, sl])
                zbuf[i, sl] = zv
                accv = accv + zv * zv
            psb[i, :] = accv
        pltpu.sync_copy(zbuf, zb_out.at[c].at[pl.ds(b, CH2)])
        pltpu.sync_copy(psb, ps_out.at[c].at[pl.ds(b, CH2)])
        return _
    lax.fori_loop(0, EC2, chunk, None)


@functools.lru_cache(maxsize=None)
def _k2():
    return pl.kernel(
        _k2_body,
        out_type=(
            jax.ShapeDtypeStruct((2, EP, HALF), _f32),
            jax.ShapeDtypeStruct((2, EP, 16), _f32),
        ),
        mesh=_mesh(),
        scratch_types=(
            pltpu.VMEM((CH2,), jnp.int32),
            pltpu.VMEM((CH2,), jnp.int32),
            pltpu.VMEM((CH2, HALF), _f32),
            pltpu.VMEM((CH2, HALF), _f32),
            pltpu.VMEM((CH2, HALF), _f32),
            pltpu.VMEM((CH2, 16), _f32),
            pltpu.VMEM((CH2, 16), _f32),
            pltpu.SemaphoreType.DMA,
            pltpu.SemaphoreType.DMA,
        ),
    )


def _k3_body(zb, ps, rowp, colp, y, dinv, zrows, zn_out, itz_out, x_out,
             xs_out, acc, ridx, cidx, zbuf, p0, p1, yb, db):
    c = lax.axis_index("c")
    t = lax.axis_index("s")
    r0 = t * RT
    pltpu.sync_copy(zrows.at[pl.ds(r0, RT)], acc.at[pl.ds(r0, RT)])
    plsc.subcore_barrier()

    def chunk(j, _):
        b = t * ET + j * CH
        pltpu.sync_copy(zb.at[c].at[pl.ds(b, CH)], zbuf)
        pltpu.sync_copy(ps.at[0].at[pl.ds(b, CH)], p0)
        pltpu.sync_copy(ps.at[1].at[pl.ds(b, CH)], p1)
        pltpu.sync_copy(rowp.at[pl.ds(b, CH)], ridx)
        pltpu.sync_copy(colp.at[pl.ds(b, CH)], cidx)
        @plsc.parallel_loop(0, CH, unroll=2)
        def one(i):
            sq = _lanesum(p0[i, :] + p1[i, :])
            sv = jnp.minimum(1.0, LAM1 * _rsqrt16(sq))
            for k in range(HALF // 16):
                sl = pl.ds(k * 16, 16)
                zbuf[i, sl] = sv * zbuf[i, sl]
        pltpu.sync_copy(zbuf, zn_out.at[c].at[pl.ds(b, CH)])
        pltpu.sync_copy(zbuf, acc.at[ridx], add=True)
        @plsc.parallel_loop(0, CH, unroll=2)
        def neg(i):
            for k in range(HALF // 16):
                sl = pl.ds(k * 16, 16)
                zbuf[i, sl] = -zbuf[i, sl]
        pltpu.sync_copy(zbuf, acc.at[cidx], add=True)
        return _
    lax.fori_loop(0, EC3, chunk, None)
    plsc.subcore_barrier()

    def dense(j, _):
        r = r0 + j * CH
        pltpu.sync_copy(acc.at[pl.ds(r, CH)], zbuf)
        pltpu.sync_copy(y.at[c].at[pl.ds(r, CH)], yb)
        pltpu.sync_copy(dinv.at[pl.ds(r, CH)], db)
        pltpu.sync_copy(zbuf, itz_out.at[c].at[pl.ds(r, CH)])
        @plsc.parallel_loop(0, CH, unroll=2)
        def one(i):
            dv = db[i, :]
            gd = GAMMA * dv
            for k in range(HALF // 16):
                sl = pl.ds(k * 16, 16)
                xv = yb[i, sl] - gd * zbuf[i, sl]
                yb[i, sl] = xv
                zbuf[i, sl] = dv * xv
        pltpu.sync_copy(yb, x_out.at[c].at[pl.ds(r, CH)])
        pltpu.sync_copy(zbuf, xs_out.at[c].at[pl.ds(r, CH)])
        return _
    lax.fori_loop(0, NDCH, dense, None)


@functools.lru_cache(maxsize=None)
def _k3():
    return pl.kernel(
        _k3_body,
        out_type=(
            jax.ShapeDtypeStruct((2, EP, HALF), _f32),
            jax.ShapeDtypeStruct((2, NP, HALF), _f32),
            jax.ShapeDtypeStruct((2, NP, HALF), _f32),
            jax.ShapeDtypeStruct((2, NP, HALF), _f32),
        ),
        mesh=_mesh(),
        scratch_types=(
            pltpu.VMEM_SHARED((NP, HALF), _f32),
            pltpu.VMEM((CH,), jnp.int32),
            pltpu.VMEM((CH,), jnp.int32),
            pltpu.VMEM((CH, HALF), _f32),
            pltpu.VMEM((CH, 16), _f32),
            pltpu.VMEM((CH, 16), _f32),
            pltpu.VMEM((CH, HALF), _f32),
            pltpu.VMEM((CH, 16), _f32),
        ),
    )


def kernel(feat, edge_index):
    src = edge_index[0]
    dst = edge_index[1]
    loops = jnp.arange(N, dtype=jnp.int32)
    srcl = jnp.concatenate([src, loops])
    dstl = jnp.concatenate([dst, loops])
    padfill = jnp.full((E1P - E1,), DUMMY, jnp.int32)
    srcl_p = jnp.concatenate([srcl, padfill])
    dstl_p = jnp.concatenate([dstl, padfill])
    mask = src > dst
    epad = jnp.full((EP - E,), DUMMY, jnp.int32)
    row_p = jnp.concatenate([src, epad])
    col_p = jnp.concatenate([dst, epad])
    bm = jnp.where(mask, _f32(BETA), _f32(0.0))
    bm_p = jnp.concatenate([bm, jnp.zeros((EP - E,), _f32)])
    bm16 = jnp.broadcast_to(bm_p[:, None], (EP, 16))
    feat2 = feat.reshape(N, 2, HALF).transpose(1, 0, 2)
    feat2 = jnp.pad(feat2, ((0, 0), (0, NP - N), (0, 0)))
    zrows = jnp.zeros((NP, HALF), _f32)

    dinv2, xs = _k0()(dstl_p, feat2)
    dinv = dinv2[0]
    z = jnp.zeros((2, EP, HALF), _f32)
    itz = jnp.zeros((2, NP, HALF), _f32)
    x2 = None
    for _ in range(KITER):
        y, xbs = _k1()(xs, feat2, itz, dinv, zrows, srcl_p, dstl_p)
        zbv, ps = _k2()(xbs, z, row_p, col_p, bm16)
        z, itz, x2, xs = _k3()(zbv, ps, row_p, col_p, y, dinv, zrows)
    x = x2[:, :N, :].transpose(1, 0, 2).reshape(N, D)
    return x


# speedup vs baseline: 1.3910x; 1.3910x over previous
import functools

import jax
import jax.numpy as jnp
from jax import lax
from jax.experimental import pallas as pl
from jax.experimental.pallas import tpu as pltpu
from jax.experimental.pallas import tpu_sc as plsc

N = 10000
D = 256
E = 160000
KITER = 3
LAM1 = 3.0
GAMMA = 1.0 / (1.0 + 3.0)
BETA = 1.0 / (2.0 * GAMMA)

NSC = 16
HALF = 128
CH = 64
CH2 = 128
RT = 640
NP = NSC * RT
NDCH = RT // CH
DUMMY = N

E1 = E + N
E1C = -(-E1 // (NSC * CH))
E1T = E1C * CH
E1P = NSC * E1T
EC2 = -(-E // (NSC * CH2))
ET = EC2 * CH2
EP = NSC * ET

_f32 = jnp.float32


def _rsqrt16(x):
    xi = lax.bitcast_convert_type(x, jnp.int32)
    yi = jnp.int32(0x5F3759DF) - (xi >> 1)
    y = lax.bitcast_convert_type(yi, _f32)
    for _ in range(4):
        y = y * (1.5 - 0.5 * x * y * y)
    return y


def _lanesum(v):
    lanes = lax.iota(jnp.int32, 16)
    for s in (8, 4, 2, 1):
        v = v + v.at[lanes ^ s].get(mode="promise_in_bounds")
    return v


def _mesh():
    return plsc.VectorSubcoreMesh(core_axis_name="c", subcore_axis_name="s")


def _k0_body(dstl, feat2, dinv_out, xs_out, accd, didx, fb, xb, db):
    c = lax.axis_index("c")
    t = lax.axis_index("s")
    r0 = t * RT
    def fill(i, _):
        for k in range(HALF // 16):
            sl = pl.ds(k * 16, 16)
            fb[i, sl] = jnp.zeros((16,), _f32)
            xb[i, sl] = jnp.ones((16,), _f32)
        return _
    lax.fori_loop(0, CH, fill, None)
    def zloop(j, _):
        pltpu.sync_copy(fb, accd.at[pl.ds(r0 + j * CH, CH)])
        return _
    lax.fori_loop(0, NDCH, zloop, None)
    plsc.subcore_barrier()
    def sloop(j, _):
        b = t * E1T + j * CH
        pltpu.sync_copy(dstl.at[pl.ds(b, CH)], didx)
        pltpu.sync_copy(xb, accd.at[didx], add=True)
        return _
    lax.fori_loop(0, E1C, sloop, None)
    plsc.subcore_barrier()
    def rloop(j, _):
        ridx = r0 + j * CH
        pltpu.sync_copy(accd.at[pl.ds(ridx, CH)], fb)
        def one(i, _2):
            d = jnp.maximum(fb[i, pl.ds(0, 16)], 1.0)
            db[i, :] = _rsqrt16(d)
            return _2
        lax.fori_loop(0, CH, one, None)
        pltpu.sync_copy(db, dinv_out.at[c].at[pl.ds(ridx, CH)])
        return _
    lax.fori_loop(0, NDCH, rloop, None)

    def xloop(j, _):
        ridx = r0 + j * CH
        pltpu.sync_copy(feat2.at[c].at[pl.ds(ridx, CH)], fb)
        pltpu.sync_copy(dinv_out.at[c].at[pl.ds(ridx, CH)], db)
        def one(i, _2):
            dv = db[i, :]
            for k in range(HALF // 16):
                sl = pl.ds(k * 16, 16)
                xb[i, sl] = dv * fb[i, sl]
            return _2
        lax.fori_loop(0, CH, one, None)
        pltpu.sync_copy(xb, xs_out.at[c].at[pl.ds(ridx, CH)])
        return _
    lax.fori_loop(0, NDCH, xloop, None)


@functools.lru_cache(maxsize=None)
def _k0():
    return pl.kernel(
        _k0_body,
        out_type=(
            jax.ShapeDtypeStruct((2, NP, 16), _f32),
            jax.ShapeDtypeStruct((2, NP, HALF), _f32),
        ),
        mesh=_mesh(),
        scratch_types=(
            pltpu.VMEM_SHARED((NP, HALF), _f32),
            pltpu.VMEM((CH,), jnp.int32),
            pltpu.VMEM((CH, HALF), _f32),
            pltpu.VMEM((CH, HALF), _f32),
            pltpu.VMEM((CH, 16), _f32),
        ),
    )


def _k1_body(xs, feat2, itz, dinv, zrows, srcl, dstl, y_out, xbs_out,
             acc, sidx, didx, ab, fb, ib, db, sem):
    c = lax.axis_index("c")
    t = lax.axis_index("s")
    r0 = t * RT
    pltpu.sync_copy(zrows.at[pl.ds(r0, RT)], acc.at[pl.ds(r0, RT)])
    pltpu.sync_copy(srcl.at[t], sidx)
    plsc.subcore_barrier()

    def chunk(j, _):
        pltpu.sync_copy(dstl.at[t, pl.ds(j * CH, CH)], didx)
        cp = pltpu.async_copy(xs.at[c].at[sidx.at[pl.ds(j * CH, CH)]], ab, sem)
        cp.wait()
        pltpu.sync_copy(ab, acc.at[didx], add=True)
        return _
    lax.fori_loop(0, E1C, chunk, None)
    plsc.subcore_barrier()

    def dense(j, _):
        r = r0 + j * CH
        pltpu.sync_copy(acc.at[pl.ds(r, CH)], ab)
        pltpu.sync_copy(feat2.at[c].at[pl.ds(r, CH)], fb)
        pltpu.sync_copy(itz.at[c].at[pl.ds(r, CH)], ib)
        pltpu.sync_copy(dinv.at[pl.ds(r * 16, CH * 16)], db)
        @plsc.parallel_loop(0, CH, unroll=2)
        def one(i):
            dv = db[pl.ds(i * 16, 16)]
            gdd = (GAMMA * dv) * dv
            for k in range(HALF // 16):
                sl = pl.ds(k * 16, 16)
                yv = GAMMA * fb[i, sl] + (1.0 - GAMMA) * (dv * ab[i, sl])
                fb[i, sl] = yv
                ab[i, sl] = dv * yv - gdd * ib[i, sl]
        pltpu.sync_copy(fb, y_out.at[c].at[pl.ds(r, CH)])
        pltpu.sync_copy(ab, xbs_out.at[c].at[pl.ds(r, CH)])
        return _
    lax.fori_loop(0, NDCH, dense, None)


@functools.lru_cache(maxsize=None)
def _k1():
    return pl.kernel(
        _k1_body,
        out_type=(
            jax.ShapeDtypeStruct((2, NP, HALF), _f32),
            jax.ShapeDtypeStruct((2, NP, HALF), _f32),
        ),
        mesh=_mesh(),
        scratch_types=(
            pltpu.VMEM_SHARED((NP, HALF), _f32),
            pltpu.VMEM((E1T,), jnp.int32),
            pltpu.VMEM((CH,), jnp.int32),
            pltpu.VMEM((CH, HALF), _f32),
            pltpu.VMEM((CH, HALF), _f32),
            pltpu.VMEM((CH, HALF), _f32),
            pltpu.VMEM((CH * 16,), _f32),
            pltpu.SemaphoreType.DMA,
        ),
    )


def _k2_body(xbs, z, rowp, colp, bm, zb_out, ps_out,
             ridx, cidx, ab, bb, zbuf, psb, bmb, sem, sem2):
    c = lax.axis_index("c")
    t = lax.axis_index("s")
    pltpu.sync_copy(rowp.at[t], ridx)
    pltpu.sync_copy(colp.at[t], cidx)

    def chunk(j, _):
        b = t * ET + j * CH2
        cpa = pltpu.async_copy(
            xbs.at[c].at[ridx.at[pl.ds(j * CH2, CH2)]], ab, sem)
        cpb = pltpu.async_copy(
            xbs.at[c].at[cidx.at[pl.ds(j * CH2, CH2)]], bb, sem2)
        pltpu.sync_copy(z.at[c].at[pl.ds(b, CH2)], zbuf)
        pltpu.sync_copy(bm.at[pl.ds(b * 16, CH2 * 16)], bmb)
        cpa.wait()
        cpb.wait()
        @plsc.parallel_loop(0, CH2, unroll=2)
        def one(i):
            mv = bmb[pl.ds(i * 16, 16)]
            accv = jnp.zeros((16,), _f32)
            for k in range(HALF // 16):
                sl = pl.ds(k * 16, 16)
                zv = zbuf[i, sl] + mv * (ab[i, sl] - bb[i, sl])
                zbuf[i, sl] = zv
                accv = accv + zv * zv
            psb[pl.ds(i * 16, 16)] = accv
        pltpu.sync_copy(zbuf, zb_out.at[c].at[pl.ds(b, CH2)])
        pltpu.sync_copy(psb, ps_out.at[c].at[pl.ds(b * 16, CH2 * 16)])
        return _
    lax.fori_loop(0, EC2, chunk, None)


@functools.lru_cache(maxsize=None)
def _k2():
    return pl.kernel(
        _k2_body,
        out_type=(
            jax.ShapeDtypeStruct((2, EP, HALF), _f32),
            jax.ShapeDtypeStruct((2, EP * 16), _f32),
        ),
        mesh=_mesh(),
        scratch_types=(
            pltpu.VMEM((ET,), jnp.int32),
            pltpu.VMEM((ET,), jnp.int32),
            pltpu.VMEM((CH2, HALF), _f32),
            pltpu.VMEM((CH2, HALF), _f32),
            pltpu.VMEM((CH2, HALF), _f32),
            pltpu.VMEM((CH2 * 16,), _f32),
            pltpu.VMEM((CH2 * 16,), _f32),
            pltpu.SemaphoreType.DMA,
            pltpu.SemaphoreType.DMA,
        ),
    )


def _k3_body(zb, ps, rc, y, dinv, zrows, zn_out, itz_out, x_out, xs_out,
             acc, rcb, zbuf, p0, p1, db):
    c = lax.axis_index("c")
    t = lax.axis_index("s")
    r0 = t * RT
    pltpu.sync_copy(zrows.at[pl.ds(r0, RT)], acc.at[pl.ds(r0, RT)])
    plsc.subcore_barrier()

    def chunk(j, _):
        b = t * ET + j * CH2
        pltpu.sync_copy(zb.at[c].at[pl.ds(b, CH2)], zbuf)
        pltpu.sync_copy(ps.at[0].at[pl.ds(b * 16, CH2 * 16)], p0)
        pltpu.sync_copy(ps.at[1].at[pl.ds(b * 16, CH2 * 16)], p1)
        pltpu.sync_copy(rc.at[t].at[j], rcb)
        @plsc.parallel_loop(0, CH2, unroll=2)
        def one(i):
            sl16 = pl.ds(i * 16, 16)
            sq = _lanesum(p0[sl16] + p1[sl16])
            sv = jnp.minimum(1.0, LAM1 * _rsqrt16(sq))
            for k in range(HALF // 16):
                sl = pl.ds(k * 16, 16)
                zbuf[i, sl] = sv * zbuf[i, sl]
        pltpu.sync_copy(zbuf, zn_out.at[c].at[pl.ds(b, CH2)])
        pltpu.sync_copy(zbuf, acc.at[rcb.at[0]], add=True)
        @plsc.parallel_loop(0, CH2, unroll=2)
        def neg(i):
            for k in range(HALF // 16):
                sl = pl.ds(k * 16, 16)
                zbuf[i, sl] = -zbuf[i, sl]
        pltpu.sync_copy(zbuf, acc.at[rcb.at[1]], add=True)
        return _
    lax.fori_loop(0, EC2, chunk, None)
    plsc.subcore_barrier()

    def dense(j, _):
        r = r0 + j * CH
        pltpu.sync_copy(acc.at[pl.ds(r, CH)], zbuf.at[pl.ds(0, CH)])
        pltpu.sync_copy(y.at[c].at[pl.ds(r, CH)], zbuf.at[pl.ds(CH, CH)])
        pltpu.sync_copy(dinv.at[pl.ds(r * 16, CH * 16)], db)
        pltpu.sync_copy(zbuf.at[pl.ds(0, CH)], itz_out.at[c].at[pl.ds(r, CH)])
        @plsc.parallel_loop(0, CH, unroll=2)
        def one(i):
            dv = db[pl.ds(i * 16, 16)]
            gd = GAMMA * dv
            for k in range(HALF // 16):
                sl = pl.ds(k * 16, 16)
                xv = zbuf[CH + i, sl] - gd * zbuf[i, sl]
                zbuf[CH + i, sl] = xv
                zbuf[i, sl] = dv * xv
        pltpu.sync_copy(zbuf.at[pl.ds(CH, CH)], x_out.at[c].at[pl.ds(r, CH)])
        pltpu.sync_copy(zbuf.at[pl.ds(0, CH)], xs_out.at[c].at[pl.ds(r, CH)])
        return _
    lax.fori_loop(0, NDCH, dense, None)


@functools.lru_cache(maxsize=None)
def _k3():
    return pl.kernel(
        _k3_body,
        out_type=(
            jax.ShapeDtypeStruct((2, EP, HALF), _f32),
            jax.ShapeDtypeStruct((2, NP, HALF), _f32),
            jax.ShapeDtypeStruct((2, NP, HALF), _f32),
            jax.ShapeDtypeStruct((2, NP, HALF), _f32),
        ),
        mesh=_mesh(),
        scratch_types=(
            pltpu.VMEM_SHARED((NP, HALF), _f32),
            pltpu.VMEM((2, CH2), jnp.int32),
            pltpu.VMEM((2 * CH, HALF), _f32),
            pltpu.VMEM((CH2 * 16,), _f32),
            pltpu.VMEM((CH2 * 16,), _f32),
            pltpu.VMEM((CH * 16,), _f32),
        ),
    )


def kernel(feat, edge_index):
    src = edge_index[0]
    dst = edge_index[1]
    loops = jnp.arange(N, dtype=jnp.int32)
    srcl = jnp.concatenate([src, loops])
    dstl = jnp.concatenate([dst, loops])
    padfill = jnp.full((E1P - E1,), DUMMY, jnp.int32)
    srcl_p = jnp.concatenate([srcl, padfill])
    dstl_p = jnp.concatenate([dstl, padfill])
    mask = src > dst
    epad = jnp.full((EP - E,), DUMMY, jnp.int32)
    row_p = jnp.concatenate([src, epad])
    col_p = jnp.concatenate([dst, epad])
    bm = jnp.where(mask, _f32(BETA), _f32(0.0))
    bm_p = jnp.concatenate([bm, jnp.zeros((EP - E,), _f32)])
    bm16 = jnp.broadcast_to(bm_p[:, None], (EP, 16)).reshape(EP * 16)
    feat2 = feat.reshape(N, 2, HALF).transpose(1, 0, 2)
    feat2 = jnp.pad(feat2, ((0, 0), (0, NP - N), (0, 0)))
    zrows = jnp.zeros((NP, HALF), _f32)

    srcl_r = srcl_p.reshape(NSC, E1T)
    dstl_r = dstl_p.reshape(NSC, E1T)
    row_r = row_p.reshape(NSC, ET)
    col_r = col_p.reshape(NSC, ET)
    rc3 = jnp.stack([row_p.reshape(NSC, EC2, CH2),
                     col_p.reshape(NSC, EC2, CH2)], axis=2)

    dinv2, xs = _k0()(dstl_p, feat2)
    dinv = dinv2[0].reshape(NP * 16)
    z = jnp.zeros((2, EP, HALF), _f32)
    itz = jnp.zeros((2, NP, HALF), _f32)
    x2 = None
    for _ in range(KITER):
        y, xbs = _k1()(xs, feat2, itz, dinv, zrows, srcl_r, dstl_r)
        zbv, ps = _k2()(xbs, z, row_r, col_r, bm16)
        z, itz, x2, xs = _k3()(zbv, ps, rc3, y, dinv, zrows)
    x = x2[:, :N, :].transpose(1, 0, 2).reshape(N, D)
    return x


# speedup vs baseline: 1.4440x; 1.0381x over previous
import functools

import jax
import jax.numpy as jnp
from jax import lax
from jax.experimental import pallas as pl
from jax.experimental.pallas import tpu as pltpu
from jax.experimental.pallas import tpu_sc as plsc

N = 10000
D = 256
E = 160000
KITER = 3
LAM1 = 3.0
GAMMA = 1.0 / (1.0 + 3.0)
BETA = 1.0 / (2.0 * GAMMA)

NSC = 16
HALF = 128
CH = 64
CH2 = 128
RT = 640
NP = NSC * RT
NDCH = RT // CH
DUMMY = N

E1 = E + N
E1C = 2 * (-(-E1 // (NSC * CH * 2)))
E1T = E1C * CH
E1P = NSC * E1T
EC2 = -(-E // (NSC * CH2))
ET = EC2 * CH2
EP = NSC * ET

_f32 = jnp.float32


def _rsqrt16(x):
    xi = lax.bitcast_convert_type(x, jnp.int32)
    yi = jnp.int32(0x5F3759DF) - (xi >> 1)
    y = lax.bitcast_convert_type(yi, _f32)
    for _ in range(4):
        y = y * (1.5 - 0.5 * x * y * y)
    return y


def _lanesum(v):
    lanes = lax.iota(jnp.int32, 16)
    for s in (8, 4, 2, 1):
        v = v + v.at[lanes ^ s].get(mode="promise_in_bounds")
    return v


def _mesh():
    return plsc.VectorSubcoreMesh(core_axis_name="c", subcore_axis_name="s")


def _k0_body(dstl, feat2, dinv_out, xs_out, accd, didx, fb, xb, db):
    c = lax.axis_index("c")
    t = lax.axis_index("s")
    r0 = t * RT
    def fill(i, _):
        for k in range(HALF // 16):
            sl = pl.ds(k * 16, 16)
            fb[i, sl] = jnp.zeros((16,), _f32)
            xb[i, sl] = jnp.ones((16,), _f32)
        return _
    lax.fori_loop(0, CH, fill, None)
    def zloop(j, _):
        pltpu.sync_copy(fb, accd.at[pl.ds(r0 + j * CH, CH)])
        return _
    lax.fori_loop(0, NDCH, zloop, None)
    plsc.subcore_barrier()
    def sloop(j, _):
        b = t * E1T + j * CH
        pltpu.sync_copy(dstl.at[pl.ds(b, CH)], didx)
        pltpu.sync_copy(xb, accd.at[didx], add=True)
        return _
    lax.fori_loop(0, E1C, sloop, None)
    plsc.subcore_barrier()
    def rloop(j, _):
        ridx = r0 + j * CH
        pltpu.sync_copy(accd.at[pl.ds(ridx, CH)], fb)
        def one(i, _2):
            d = jnp.maximum(fb[i, pl.ds(0, 16)], 1.0)
            db[i, :] = _rsqrt16(d)
            return _2
        lax.fori_loop(0, CH, one, None)
        pltpu.sync_copy(db, dinv_out.at[c].at[pl.ds(ridx, CH)])
        return _
    lax.fori_loop(0, NDCH, rloop, None)

    def xloop(j, _):
        ridx = r0 + j * CH
        pltpu.sync_copy(feat2.at[c].at[pl.ds(ridx, CH)], fb)
        pltpu.sync_copy(dinv_out.at[c].at[pl.ds(ridx, CH)], db)
        def one(i, _2):
            dv = db[i, :]
            for k in range(HALF // 16):
                sl = pl.ds(k * 16, 16)
                xb[i, sl] = dv * fb[i, sl]
            return _2
        lax.fori_loop(0, CH, one, None)
        pltpu.sync_copy(xb, xs_out.at[c].at[pl.ds(ridx, CH)])
        return _
    lax.fori_loop(0, NDCH, xloop, None)


@functools.lru_cache(maxsize=None)
def _k0():
    return pl.kernel(
        _k0_body,
        out_type=(
            jax.ShapeDtypeStruct((2, NP, 16), _f32),
            jax.ShapeDtypeStruct((2, NP, HALF), _f32),
        ),
        mesh=_mesh(),
        scratch_types=(
            pltpu.VMEM_SHARED((NP, HALF), _f32),
            pltpu.VMEM((CH,), jnp.int32),
            pltpu.VMEM((CH, HALF), _f32),
            pltpu.VMEM((CH, HALF), _f32),
            pltpu.VMEM((CH, 16), _f32),
        ),
    )


def _k1_body(xs, feat2, itz, dinv, zrows, srcl, dstl, y_out, xbs_out,
             acc, sidx, didx, didx2, ab, fb, ib, db, sem, sem2):
    c = lax.axis_index("c")
    t = lax.axis_index("s")
    r0 = t * RT
    pltpu.sync_copy(zrows.at[pl.ds(r0, RT)], acc.at[pl.ds(r0, RT)])
    pltpu.sync_copy(srcl.at[t], sidx)
    plsc.subcore_barrier()

    def chunk(jj, _):
        j0 = jj * 2
        j1 = j0 + 1
        cp0 = pltpu.async_copy(
            xs.at[c].at[sidx.at[pl.ds(j0 * CH, CH)]], ab, sem)
        cp1 = pltpu.async_copy(
            xs.at[c].at[sidx.at[pl.ds(j1 * CH, CH)]], fb, sem2)
        pltpu.sync_copy(dstl.at[t, pl.ds(j0 * CH, CH)], didx)
        pltpu.sync_copy(dstl.at[t, pl.ds(j1 * CH, CH)], didx2)
        cp0.wait()
        pltpu.sync_copy(ab, acc.at[didx], add=True)
        cp1.wait()
        pltpu.sync_copy(fb, acc.at[didx2], add=True)
        return _
    lax.fori_loop(0, E1C // 2, chunk, None)
    plsc.subcore_barrier()

    def dense(j, _):
        r = r0 + j * CH
        pltpu.sync_copy(acc.at[pl.ds(r, CH)], ab)
        pltpu.sync_copy(feat2.at[c].at[pl.ds(r, CH)], fb)
        pltpu.sync_copy(itz.at[c].at[pl.ds(r, CH)], ib)
        pltpu.sync_copy(dinv.at[pl.ds(r * 16, CH * 16)], db)
        @plsc.parallel_loop(0, CH, unroll=2)
        def one(i):
            dv = db[pl.ds(i * 16, 16)]
            gdd = (GAMMA * dv) * dv
            for k in range(HALF // 16):
                sl = pl.ds(k * 16, 16)
                yv = GAMMA * fb[i, sl] + (1.0 - GAMMA) * (dv * ab[i, sl])
                fb[i, sl] = yv
                ab[i, sl] = dv * yv - gdd * ib[i, sl]
        pltpu.sync_copy(fb, y_out.at[c].at[pl.ds(r, CH)])
        pltpu.sync_copy(ab, xbs_out.at[c].at[pl.ds(r, CH)])
        return _
    lax.fori_loop(0, NDCH, dense, None)


@functools.lru_cache(maxsize=None)
def _k1():
    return pl.kernel(
        _k1_body,
        out_type=(
            jax.ShapeDtypeStruct((2, NP, HALF), _f32),
            jax.ShapeDtypeStruct((2, NP, HALF), _f32),
        ),
        mesh=_mesh(),
        scratch_types=(
            pltpu.VMEM_SHARED((NP, HALF), _f32),
            pltpu.VMEM((E1T,), jnp.int32),
            pltpu.VMEM((CH,), jnp.int32),
            pltpu.VMEM((CH,), jnp.int32),
            pltpu.VMEM((CH, HALF), _f32),
            pltpu.VMEM((CH, HALF), _f32),
            pltpu.VMEM((CH, HALF), _f32),
            pltpu.VMEM((CH * 16,), _f32),
            pltpu.SemaphoreType.DMA,
            pltpu.SemaphoreType.DMA,
        ),
    )


def _k2_body(xbs, z, rowp, colp, bm, zb_out, ps_out,
             ridx, cidx, ab, bb, zbuf, psb, bmb, sem, sem2):
    c = lax.axis_index("c")
    t = lax.axis_index("s")
    pltpu.sync_copy(rowp.at[t], ridx)
    pltpu.sync_copy(colp.at[t], cidx)

    def chunk(j, _):
        b = t * ET + j * CH2
        cpa = pltpu.async_copy(
            xbs.at[c].at[ridx.at[pl.ds(j * CH2, CH2)]], ab, sem)
        cpb = pltpu.async_copy(
            xbs.at[c].at[cidx.at[pl.ds(j * CH2, CH2)]], bb, sem2)
        pltpu.sync_copy(z.at[c].at[pl.ds(b, CH2)], zbuf)
        pltpu.sync_copy(bm.at[pl.ds(b * 16, CH2 * 16)], bmb)
        cpa.wait()
        cpb.wait()
        @plsc.parallel_loop(0, CH2, unroll=2)
        def one(i):
            mv = bmb[pl.ds(i * 16, 16)]
            accv = jnp.zeros((16,), _f32)
            for k in range(HALF // 16):
                sl = pl.ds(k * 16, 16)
                zv = zbuf[i, sl] + mv * (ab[i, sl] - bb[i, sl])
                zbuf[i, sl] = zv
                accv = accv + zv * zv
            psb[pl.ds(i * 16, 16)] = accv
        pltpu.sync_copy(zbuf, zb_out.at[c].at[pl.ds(b, CH2)])
        pltpu.sync_copy(psb, ps_out.at[c].at[pl.ds(b * 16, CH2 * 16)])
        return _
    lax.fori_loop(0, EC2, chunk, None)


@functools.lru_cache(maxsize=None)
def _k2():
    return pl.kernel(
        _k2_body,
        out_type=(
            jax.ShapeDtypeStruct((2, EP, HALF), _f32),
            jax.ShapeDtypeStruct((2, EP * 16), _f32),
        ),
        mesh=_mesh(),
        scratch_types=(
            pltpu.VMEM((ET,), jnp.int32),
            pltpu.VMEM((ET,), jnp.int32),
            pltpu.VMEM((CH2, HALF), _f32),
            pltpu.VMEM((CH2, HALF), _f32),
            pltpu.VMEM((CH2, HALF), _f32),
            pltpu.VMEM((CH2 * 16,), _f32),
            pltpu.VMEM((CH2 * 16,), _f32),
            pltpu.SemaphoreType.DMA,
            pltpu.SemaphoreType.DMA,
        ),
    )


def _k3_body(zb, ps, rc, y, dinv, zrows, zn_out, itz_out, x_out, xs_out,
             acc, rcb, zbuf, p0, p1, db):
    c = lax.axis_index("c")
    t = lax.axis_index("s")
    r0 = t * RT
    pltpu.sync_copy(zrows.at[pl.ds(r0, RT)], acc.at[pl.ds(r0, RT)])
    plsc.subcore_barrier()

    def chunk(j, _):
        b = t * ET + j * CH2
        pltpu.sync_copy(zb.at[c].at[pl.ds(b, CH2)], zbuf)
        pltpu.sync_copy(ps.at[0].at[pl.ds(b * 16, CH2 * 16)], p0)
        pltpu.sync_copy(ps.at[1].at[pl.ds(b * 16, CH2 * 16)], p1)
        pltpu.sync_copy(rc.at[t].at[j], rcb)
        @plsc.parallel_loop(0, CH2, unroll=2)
        def one(i):
            sl16 = pl.ds(i * 16, 16)
            sq = _lanesum(p0[sl16] + p1[sl16])
            sv = jnp.minimum(1.0, LAM1 * _rsqrt16(sq))
            for k in range(HALF // 16):
                sl = pl.ds(k * 16, 16)
                zbuf[i, sl] = sv * zbuf[i, sl]
        pltpu.sync_copy(zbuf, zn_out.at[c].at[pl.ds(b, CH2)])
        pltpu.sync_copy(zbuf, acc.at[rcb.at[0]], add=True)
        @plsc.parallel_loop(0, CH2, unroll=2)
        def neg(i):
            for k in range(HALF // 16):
                sl = pl.ds(k * 16, 16)
                zbuf[i, sl] = -zbuf[i, sl]
        pltpu.sync_copy(zbuf, acc.at[rcb.at[1]], add=True)
        return _
    lax.fori_loop(0, EC2, chunk, None)
    plsc.subcore_barrier()

    def dense(j, _):
        r = r0 + j * CH
        pltpu.sync_copy(acc.at[pl.ds(r, CH)], zbuf.at[pl.ds(0, CH)])
        pltpu.sync_copy(y.at[c].at[pl.ds(r, CH)], zbuf.at[pl.ds(CH, CH)])
        pltpu.sync_copy(dinv.at[pl.ds(r * 16, CH * 16)], db)
        pltpu.sync_copy(zbuf.at[pl.ds(0, CH)], itz_out.at[c].at[pl.ds(r, CH)])
        @plsc.parallel_loop(0, CH, unroll=2)
        def one(i):
            dv = db[pl.ds(i * 16, 16)]
            gd = GAMMA * dv
            for k in range(HALF // 16):
                sl = pl.ds(k * 16, 16)
                xv = zbuf[CH + i, sl] - gd * zbuf[i, sl]
                zbuf[CH + i, sl] = xv
                zbuf[i, sl] = dv * xv
        pltpu.sync_copy(zbuf.at[pl.ds(CH, CH)], x_out.at[c].at[pl.ds(r, CH)])
        pltpu.sync_copy(zbuf.at[pl.ds(0, CH)], xs_out.at[c].at[pl.ds(r, CH)])
        return _
    lax.fori_loop(0, NDCH, dense, None)


@functools.lru_cache(maxsize=None)
def _k3():
    return pl.kernel(
        _k3_body,
        out_type=(
            jax.ShapeDtypeStruct((2, EP, HALF), _f32),
            jax.ShapeDtypeStruct((2, NP, HALF), _f32),
            jax.ShapeDtypeStruct((2, NP, HALF), _f32),
            jax.ShapeDtypeStruct((2, NP, HALF), _f32),
        ),
        mesh=_mesh(),
        scratch_types=(
            pltpu.VMEM_SHARED((NP, HALF), _f32),
            pltpu.VMEM((2, CH2), jnp.int32),
            pltpu.VMEM((2 * CH, HALF), _f32),
            pltpu.VMEM((CH2 * 16,), _f32),
            pltpu.VMEM((CH2 * 16,), _f32),
            pltpu.VMEM((CH * 16,), _f32),
        ),
    )


def kernel(feat, edge_index):
    src = edge_index[0]
    dst = edge_index[1]
    loops = jnp.arange(N, dtype=jnp.int32)
    srcl = jnp.concatenate([src, loops])
    dstl = jnp.concatenate([dst, loops])
    padfill = jnp.full((E1P - E1,), DUMMY, jnp.int32)
    srcl_p = jnp.concatenate([srcl, padfill])
    dstl_p = jnp.concatenate([dstl, padfill])
    mask = src > dst
    epad = jnp.full((EP - E,), DUMMY, jnp.int32)
    row_p = jnp.concatenate([src, epad])
    col_p = jnp.concatenate([dst, epad])
    bm = jnp.where(mask, _f32(BETA), _f32(0.0))
    bm_p = jnp.concatenate([bm, jnp.zeros((EP - E,), _f32)])
    bm16 = jnp.broadcast_to(bm_p[:, None], (EP, 16)).reshape(EP * 16)
    feat2 = feat.reshape(N, 2, HALF).transpose(1, 0, 2)
    feat2 = jnp.pad(feat2, ((0, 0), (0, NP - N), (0, 0)))
    zrows = jnp.zeros((NP, HALF), _f32)

    srcl_r = srcl_p.reshape(NSC, E1T)
    dstl_r = dstl_p.reshape(NSC, E1T)
    row_r = row_p.reshape(NSC, ET)
    col_r = col_p.reshape(NSC, ET)
    rc3 = jnp.stack([row_p.reshape(NSC, EC2, CH2),
                     col_p.reshape(NSC, EC2, CH2)], axis=2)

    dinv2, xs = _k0()(dstl_p, feat2)
    dinv = dinv2[0].reshape(NP * 16)
    z = jnp.zeros((2, EP, HALF), _f32)
    itz = jnp.zeros((2, NP, HALF), _f32)
    x2 = None
    for _ in range(KITER):
        y, xbs = _k1()(xs, feat2, itz, dinv, zrows, srcl_r, dstl_r)
        zbv, ps = _k2()(xbs, z, row_r, col_r, bm16)
        z, itz, x2, xs = _k3()(zbv, ps, rc3, y, dinv, zrows)
    x = x2[:, :N, :].transpose(1, 0, 2).reshape(N, D)
    return x
